# baseline (speedup 1.0000x reference)
import functools

import jax
import jax.numpy as jnp
from jax import lax
from jax.experimental import pallas as pl
from jax.experimental.pallas import tpu as pltpu
from jax.experimental.pallas import tpu_sc as plsc

B = 262144
C = 32
RES = 512
HW = RES * RES
PLANE_DIMS = ((0, 1), (0, 2), (1, 2))

NUM_WORKERS = 32
PTS_PER_W = B // NUM_WORKERS
CHUNK = 128
NCHUNKS = PTS_PER_W // CHUNK
NGROUPS = CHUNK // 16

TBLK = 8192


def _table_body(src_ref, dst_ref):
    dst_ref[...] = src_ref[...].astype(jnp.bfloat16).T


_table_call = pl.pallas_call(
    _table_body,
    out_shape=jax.ShapeDtypeStruct((HW, C), jnp.bfloat16),
    grid=(HW // TBLK,),
    in_specs=[pl.BlockSpec((C, TBLK), lambda i: (0, i))],
    out_specs=pl.BlockSpec((TBLK, C), lambda i: (i, 0)),
)


def _sc_body(xh, t0h, t1h, t2h, outh, *scr):
    xb = scr[0]
    fb = (scr[1:4], scr[4:7])
    ib = (scr[7:19], scr[19:31])
    gb = (scr[31:43], scr[43:55])
    ob = scr[55:57]
    gsem, osem = scr[57:59], scr[59:61]
    tabs = (t0h, t1h, t2h)
    wid = lax.axis_index("s") * 2 + lax.axis_index("c")
    base = wid * PTS_PER_W

    pltpu.sync_copy(xh.at[pl.ds(base, PTS_PER_W)], xb)

    def prep(jj, s):
        coff = jj * CHUNK
        iota = lax.iota(jnp.int32, 16)
        for g in range(NGROUPS):
            gs = pl.ds(g * 16, 16)
            rows = iota + (coff + g * 16)
            lo, lop = [], []
            for d in range(3):
                xd = plsc.load_gather(xb, [rows, jnp.full((16,), d, jnp.int32)])
                ix = (xd + 1.0) * 0.5 * float(RES - 1)
                ii = ix.astype(jnp.int32)
                ii = jnp.clip(ii, 0, RES - 1)
                fd = ix - ii.astype(jnp.float32)
                iip = jnp.minimum(ii + 1, RES - 1)
                fb[s][d][gs] = fd
                lo.append(ii)
                lop.append(iip)
            for p, (d0, d1) in enumerate(PLANE_DIMS):
                r0 = lo[d1] * RES
                r1 = lop[d1] * RES
                ib[s][4 * p + 0][gs] = r0 + lo[d0]
                ib[s][4 * p + 1][gs] = r0 + lop[d0]
                ib[s][4 * p + 2][gs] = r1 + lo[d0]
                ib[s][4 * p + 3][gs] = r1 + lop[d0]

    def gather_fire(s):
        for p in range(3):
            for t in range(4):
                k = 4 * p + t
                pltpu.make_async_copy(
                    tabs[p].at[ib[s][k]], gb[s][k], gsem[s]
                ).start()

    def gather_wait(s):
        for p in range(3):
            for t in range(4):
                k = 4 * p + t
                pltpu.make_async_copy(
                    tabs[p].at[ib[s][k]], gb[s][k], gsem[s]
                ).wait()

    def out_fire(jj, s):
        pltpu.make_async_copy(
            ob[s], outh.at[pl.ds(base + jj * CHUNK, CHUNK)], osem[s]
        ).start()

    def out_wait(jj, s):
        pltpu.make_async_copy(
            ob[s], outh.at[pl.ds(base + jj * CHUNK, CHUNK)], osem[s]
        ).wait()

    def compute(s):
        def pt_group(g, carry):
            gs = pl.ds(g * 16, 16)
            f0v = fb[s][0][gs]
            f1v = fb[s][1][gs]
            f2v = fb[s][2][gs]
            col2 = lax.iota(jnp.int32, 16) * 2
            for l in range(16):
                bpt = g * 16 + l
                wxs = (f0v[l], f0v[l], f1v[l])
                wys = (f1v[l], f2v[l], f2v[l])
                acc = [None, None]
                for p in range(3):
                    wx = wxs[p]
                    wy = wys[p]
                    taps = []
                    for t in range(4):
                        row = gb[s][4 * p + t][bpt, :]
                        taps.append(plsc.unpack(
                            row, format=plsc.PackFormat.INTERLEAVED))
                    for h in range(2):
                        v00, v01, v10, v11 = (taps[0][h], taps[1][h],
                                              taps[2][h], taps[3][h])
                        top = v00 + wx * (v01 - v00)
                        bot = v10 + wx * (v11 - v10)
                        f = top + wy * (bot - top)
                        acc[h] = f if acc[h] is None else acc[h] * f
                rowi = jnp.full((16,), bpt, dtype=jnp.int32)
                plsc.store_scatter(ob[s], [rowi, col2], acc[0])
                plsc.store_scatter(ob[s], [rowi, col2 + 1], acc[1])
            return carry

        lax.fori_loop(0, NGROUPS, pt_group, 0)

    prep(0, 0)
    gather_fire(0)

    def pair_body(jh, carry):
        for par in range(2):
            jj = 2 * jh + par
            s, s2 = par, 1 - par

            @pl.when(jj + 1 < NCHUNKS)
            def _():
                prep(jj + 1, s2)
                gather_fire(s2)

            gather_wait(s)

            @pl.when(jj >= 2)
            def _():
                out_wait(jj - 2, s)

            compute(s)
            out_fire(jj, s)
        return carry

    lax.fori_loop(0, NCHUNKS // 2, pair_body, 0)

    out_wait(NCHUNKS - 2, 0)
    out_wait(NCHUNKS - 1, 1)


_sc_call = functools.partial(
    pl.kernel,
    out_type=jax.ShapeDtypeStruct((B, C), jnp.float32),
    mesh=plsc.VectorSubcoreMesh(core_axis_name="c", subcore_axis_name="s"),
    compiler_params=pltpu.CompilerParams(
        use_tc_tiling_on_sc=False, needs_layout_passes=False),
    scratch_types=(
        [pltpu.VMEM((PTS_PER_W, 3), jnp.float32)]
        + [pltpu.VMEM((CHUNK,), jnp.float32) for _ in range(6)]
        + [pltpu.VMEM((CHUNK,), jnp.int32) for _ in range(24)]
        + [pltpu.VMEM((CHUNK, C), jnp.bfloat16) for _ in range(24)]
        + [pltpu.VMEM((CHUNK, C), jnp.float32) for _ in range(2)]
        + [pltpu.SemaphoreType.DMA for _ in range(4)]
    ),
)(_sc_body)


def kernel(x, plane0, plane1, plane2):
    t0 = _table_call(plane0.reshape(C, HW))
    t1 = _table_call(plane1.reshape(C, HW))
    t2 = _table_call(plane2.reshape(C, HW))
    return _sc_call(x, t0, t1, t2)


# speedup vs baseline: 1.3596x; 1.3596x over previous
import functools

import jax
import jax.numpy as jnp
from jax import lax
from jax.experimental import pallas as pl
from jax.experimental.pallas import tpu as pltpu
from jax.experimental.pallas import tpu_sc as plsc

B = 262144
C = 32
RES = 512
HW = RES * RES
PLANE_DIMS = ((0, 1), (0, 2), (1, 2))

NUM_WORKERS = 32
PTS_PER_W = B // NUM_WORKERS
CHUNK = 128
NCHUNKS = PTS_PER_W // CHUNK
NGROUPS = CHUNK // 16

TBH = 64


def _table_body(src_ref, dst_ref):
    blk = src_ref[0].astype(jnp.bfloat16).reshape(C, TBH * RES)
    dst_ref[...] = blk.T


_table_call = pl.pallas_call(
    _table_body,
    out_shape=jax.ShapeDtypeStruct((HW, C), jnp.bfloat16),
    grid=(RES // TBH,),
    in_specs=[pl.BlockSpec((1, C, TBH, RES), lambda i: (0, 0, i, 0))],
    out_specs=pl.BlockSpec((TBH * RES, C), lambda i: (i, 0)),
)


def _sc_body(xh, t0h, t1h, t2h, outh, *scr):
    xbufs = scr[0:3]
    fb = (scr[3:6], scr[6:9])
    ib = (scr[9:21], scr[21:33])
    gb = (scr[33:45], scr[45:57])
    ob = scr[57:59]
    gsem, osem = scr[59:61], scr[61:63]
    tabs = (t0h, t1h, t2h)
    wid = lax.axis_index("s") * 2 + lax.axis_index("c")
    base = wid * PTS_PER_W

    for d in range(3):
        pltpu.sync_copy(xh.at[d, pl.ds(base, PTS_PER_W)], xbufs[d])

    def prep(jj, s):
        coff = jj * CHUNK
        for g in range(NGROUPS):
            gs = pl.ds(g * 16, 16)
            lo, lop = [], []
            for d in range(3):
                xd = xbufs[d][pl.ds(coff + g * 16, 16)]
                ix = (xd + 1.0) * 0.5 * float(RES - 1)
                ii = ix.astype(jnp.int32)
                ii = jnp.clip(ii, 0, RES - 1)
                fd = ix - ii.astype(jnp.float32)
                iip = jnp.minimum(ii + 1, RES - 1)
                fb[s][d][gs] = fd
                lo.append(ii)
                lop.append(iip)
            for p, (d0, d1) in enumerate(PLANE_DIMS):
                r0 = lo[d1] * RES
                r1 = lop[d1] * RES
                ib[s][4 * p + 0][gs] = r0 + lo[d0]
                ib[s][4 * p + 1][gs] = r0 + lop[d0]
                ib[s][4 * p + 2][gs] = r1 + lo[d0]
                ib[s][4 * p + 3][gs] = r1 + lop[d0]

    def gather_fire(s):
        for p in range(3):
            for t in range(4):
                k = 4 * p + t
                pltpu.make_async_copy(
                    tabs[p].at[ib[s][k]], gb[s][k], gsem[s]
                ).start()

    def gather_wait(s):
        for p in range(3):
            for t in range(4):
                k = 4 * p + t
                pltpu.make_async_copy(
                    tabs[p].at[ib[s][k]], gb[s][k], gsem[s]
                ).wait()

    def out_fire(jj, s):
        pltpu.make_async_copy(
            ob[s], outh.at[pl.ds(base + jj * CHUNK, CHUNK)], osem[s]
        ).start()

    def out_wait(jj, s):
        pltpu.make_async_copy(
            ob[s], outh.at[pl.ds(base + jj * CHUNK, CHUNK)], osem[s]
        ).wait()

    def compute(s):
        def pt_group(g, carry):
            gs = pl.ds(g * 16, 16)
            f0v = fb[s][0][gs]
            f1v = fb[s][1][gs]
            f2v = fb[s][2][gs]
            col2 = lax.iota(jnp.int32, 16) * 2
            for l in range(16):
                bpt = g * 16 + l
                wxs = (f0v[l], f0v[l], f1v[l])
                wys = (f1v[l], f2v[l], f2v[l])
                acc = [None, None]
                for p in range(3):
                    wx = wxs[p]
                    wy = wys[p]
                    taps = []
                    for t in range(4):
                        row = gb[s][4 * p + t][bpt, :]
                        taps.append(plsc.unpack(
                            row, format=plsc.PackFormat.INTERLEAVED))
                    for h in range(2):
                        v00, v01, v10, v11 = (taps[0][h], taps[1][h],
                                              taps[2][h], taps[3][h])
                        top = v00 + wx * (v01 - v00)
                        bot = v10 + wx * (v11 - v10)
                        f = top + wy * (bot - top)
                        acc[h] = f if acc[h] is None else acc[h] * f
                rowi = jnp.full((16,), bpt, dtype=jnp.int32)
                plsc.store_scatter(ob[s], [rowi, col2], acc[0])
                plsc.store_scatter(ob[s], [rowi, col2 + 1], acc[1])
            return carry

        lax.fori_loop(0, NGROUPS, pt_group, 0)

    prep(0, 0)
    gather_fire(0)

    def pair_body(jh, carry):
        for par in range(2):
            jj = 2 * jh + par
            s, s2 = par, 1 - par

            @pl.when(jj + 1 < NCHUNKS)
            def _():
                prep(jj + 1, s2)
                gather_fire(s2)

            gather_wait(s)

            @pl.when(jj >= 2)
            def _():
                out_wait(jj - 2, s)

            compute(s)
            out_fire(jj, s)
        return carry

    lax.fori_loop(0, NCHUNKS // 2, pair_body, 0)

    out_wait(NCHUNKS - 2, 0)
    out_wait(NCHUNKS - 1, 1)


_sc_call = functools.partial(
    pl.kernel,
    out_type=jax.ShapeDtypeStruct((B, C), jnp.float32),
    mesh=plsc.VectorSubcoreMesh(core_axis_name="c", subcore_axis_name="s"),
    compiler_params=pltpu.CompilerParams(
        use_tc_tiling_on_sc=False, needs_layout_passes=False),
    scratch_types=(
        [pltpu.VMEM((PTS_PER_W,), jnp.float32) for _ in range(3)]
        + [pltpu.VMEM((CHUNK,), jnp.float32) for _ in range(6)]
        + [pltpu.VMEM((CHUNK,), jnp.int32) for _ in range(24)]
        + [pltpu.VMEM((CHUNK, C), jnp.bfloat16) for _ in range(24)]
        + [pltpu.VMEM((CHUNK, C), jnp.float32) for _ in range(2)]
        + [pltpu.SemaphoreType.DMA for _ in range(4)]
    ),
)(_sc_body)


def kernel(x, plane0, plane1, plane2):
    t0 = _table_call(plane0)
    t1 = _table_call(plane1)
    t2 = _table_call(plane2)
    return _sc_call(x.T, t0, t1, t2)


# speedup vs baseline: 1.7861x; 1.3137x over previous
import functools

import jax
import jax.numpy as jnp
from jax import lax
from jax.experimental import pallas as pl
from jax.experimental.pallas import tpu as pltpu
from jax.experimental.pallas import tpu_sc as plsc

B = 262144
C = 32
RES = 512
HW = RES * RES
PLANE_DIMS = ((0, 1), (0, 2), (1, 2))

NUM_WORKERS = 32
PTS_PER_W = B // NUM_WORKERS
CHUNK = 128
NCHUNKS = PTS_PER_W // CHUNK
NGROUPS = CHUNK // 16


def _sc_body(xh, t0h, t1h, t2h, outh, *scr):
    xb = (scr[0:3], scr[3:6])
    fb = (scr[6:9], scr[9:12])
    ib = (scr[12:24], scr[24:36])
    gb = (scr[36:48], scr[48:60])
    ob = scr[60:62]
    xsem, gsem, osem = scr[62:64], scr[64:66], scr[66:68]
    tabs = (t0h, t1h, t2h)
    wid = lax.axis_index("s") * 2 + lax.axis_index("c")
    base = wid * PTS_PER_W

    def x_fire(jj, s):
        for d in range(3):
            pltpu.make_async_copy(
                xh.at[d, pl.ds(base + jj * CHUNK, CHUNK)], xb[s][d], xsem[s]
            ).start()

    def x_wait(jj, s):
        for d in range(3):
            pltpu.make_async_copy(
                xh.at[d, pl.ds(base + jj * CHUNK, CHUNK)], xb[s][d], xsem[s]
            ).wait()

    def prep(s):
        for g in range(NGROUPS):
            gs = pl.ds(g * 16, 16)
            lo, lop = [], []
            for d in range(3):
                xd = xb[s][d][gs]
                ix = (xd + 1.0) * 0.5 * float(RES - 1)
                ii = ix.astype(jnp.int32)
                ii = jnp.clip(ii, 0, RES - 1)
                fd = ix - ii.astype(jnp.float32)
                iip = jnp.minimum(ii + 1, RES - 1)
                fb[s][d][gs] = fd
                lo.append(ii)
                lop.append(iip)
            for p, (d0, d1) in enumerate(PLANE_DIMS):
                r0 = lo[d1] * RES
                r1 = lop[d1] * RES
                ib[s][4 * p + 0][gs] = r0 + lo[d0]
                ib[s][4 * p + 1][gs] = r0 + lop[d0]
                ib[s][4 * p + 2][gs] = r1 + lo[d0]
                ib[s][4 * p + 3][gs] = r1 + lop[d0]

    def gather_fire(s):
        for p in range(3):
            for t in range(4):
                k = 4 * p + t
                pltpu.make_async_copy(
                    tabs[p].at[ib[s][k]], gb[s][k], gsem[s]
                ).start()

    def gather_wait(s):
        for p in range(3):
            for t in range(4):
                k = 4 * p + t
                pltpu.make_async_copy(
                    tabs[p].at[ib[s][k]], gb[s][k], gsem[s]
                ).wait()

    def out_fire(jj, s):
        pltpu.make_async_copy(
            ob[s], outh.at[pl.ds(base + jj * CHUNK, CHUNK)], osem[s]
        ).start()

    def out_wait(jj, s):
        pltpu.make_async_copy(
            ob[s], outh.at[pl.ds(base + jj * CHUNK, CHUNK)], osem[s]
        ).wait()

    def compute(s):
        def pt_group(g, carry):
            gs = pl.ds(g * 16, 16)
            f0v = fb[s][0][gs]
            f1v = fb[s][1][gs]
            f2v = fb[s][2][gs]
            for l in range(16):
                bpt = g * 16 + l
                wxs = (f0v[l], f0v[l], f1v[l])
                wys = (f1v[l], f2v[l], f2v[l])
                for h in range(2):
                    hs = pl.ds(h * 16, 16)
                    acc = None
                    for p in range(3):
                        wx = wxs[p]
                        wy = wys[p]
                        v00 = gb[s][4 * p + 0][bpt, hs]
                        v01 = gb[s][4 * p + 1][bpt, hs]
                        v10 = gb[s][4 * p + 2][bpt, hs]
                        v11 = gb[s][4 * p + 3][bpt, hs]
                        top = v00 + wx * (v01 - v00)
                        bot = v10 + wx * (v11 - v10)
                        f = top + wy * (bot - top)
                        acc = f if acc is None else acc * f
                    ob[s][bpt, hs] = acc
            return carry

        lax.fori_loop(0, NGROUPS, pt_group, 0)

    x_fire(0, 0)
    x_wait(0, 0)
    prep(0)
    gather_fire(0)
    x_fire(1, 1)

    def pair_body(jh, carry):
        for par in range(2):
            jj = 2 * jh + par
            s, s2 = par, 1 - par

            @pl.when(jj + 1 < NCHUNKS)
            def _():
                x_wait(jj + 1, s2)
                prep(s2)
                gather_fire(s2)

            @pl.when(jj + 2 < NCHUNKS)
            def _():
                x_fire(jj + 2, s)

            gather_wait(s)

            @pl.when(jj >= 2)
            def _():
                out_wait(jj - 2, s)

            compute(s)
            out_fire(jj, s)
        return carry

    lax.fori_loop(0, NCHUNKS // 2, pair_body, 0)

    out_wait(NCHUNKS - 2, 0)
    out_wait(NCHUNKS - 1, 1)


_sc_call = functools.partial(
    pl.kernel,
    out_type=jax.ShapeDtypeStruct((B, C), jnp.float32),
    mesh=plsc.VectorSubcoreMesh(core_axis_name="c", subcore_axis_name="s"),
    compiler_params=pltpu.CompilerParams(
        use_tc_tiling_on_sc=False, needs_layout_passes=False),
    scratch_types=(
        [pltpu.VMEM((CHUNK,), jnp.float32) for _ in range(6)]
        + [pltpu.VMEM((CHUNK,), jnp.float32) for _ in range(6)]
        + [pltpu.VMEM((CHUNK,), jnp.int32) for _ in range(24)]
        + [pltpu.VMEM((CHUNK, C), jnp.float32) for _ in range(24)]
        + [pltpu.VMEM((CHUNK, C), jnp.float32) for _ in range(2)]
        + [pltpu.SemaphoreType.DMA for _ in range(6)]
    ),
)(_sc_body)


def kernel(x, plane0, plane1, plane2):
    t0 = plane0[0].transpose(1, 2, 0).reshape(HW, C)
    t1 = plane1[0].transpose(1, 2, 0).reshape(HW, C)
    t2 = plane2[0].transpose(1, 2, 0).reshape(HW, C)
    return _sc_call(x.T, t0, t1, t2)
